# baseline (speedup 1.0000x reference)
import functools

import jax
import jax.numpy as jnp
from jax import lax
from jax.experimental import pallas as pl
from jax.experimental.pallas import tpu as pltpu
from jax.experimental.pallas import tpu_sc as plsc

VERTEXES = 100000
EDGES = 1600000

NUM_CORES = 2
NUM_SUBCORES = 16
NUM_TILES = NUM_CORES * NUM_SUBCORES

V_SLICE = 6256
V_PAD = V_SLICE * NUM_SUBCORES

CHUNK = 10000
EDGES_PER_TILE = EDGES // NUM_TILES
TILE_CHUNKS = EDGES_PER_TILE // CHUNK

_LANES = 16

_MESH = plsc.VectorSubcoreMesh(core_axis_name="c", subcore_axis_name="s",
                               num_cores=NUM_CORES, num_subcores=NUM_SUBCORES)


@functools.partial(
    pl.kernel,
    out_type=jax.ShapeDtypeStruct((NUM_CORES * 2 * V_PAD,), jnp.float32),
    mesh=_MESH,
    scratch_types=(
        pltpu.VMEM_SHARED((V_PAD,), jnp.float32),
        pltpu.VMEM_SHARED((V_PAD,), jnp.float32),
        pltpu.VMEM((CHUNK,), jnp.int32),
        pltpu.VMEM((CHUNK,), jnp.int32),
        pltpu.VMEM((CHUNK,), jnp.float32),
        pltpu.VMEM((V_SLICE,), jnp.float32),
        pltpu.SemaphoreType.DMA,
    ),
)
def _count_partials(recv_hbm, send_hbm, part_hbm, cnt_fwd, cnt_bwd,
                    idx_f, idx_b, ones_buf, zero_buf, sem):
    c = lax.axis_index("c")
    s = lax.axis_index("s")

    def _fill(i, _):
        ones_buf[pl.ds(i * _LANES, _LANES)] = jnp.full((_LANES,), 1.0, jnp.float32)
        return 0
    lax.fori_loop(0, CHUNK // _LANES, _fill, 0)

    def _zero(i, _):
        zero_buf[pl.ds(i * _LANES, _LANES)] = jnp.zeros((_LANES,), jnp.float32)
        return 0
    lax.fori_loop(0, V_SLICE // _LANES, _zero, 0)
    voff = s * V_SLICE
    pltpu.sync_copy(zero_buf, cnt_fwd.at[pl.ds(voff, V_SLICE)])
    pltpu.sync_copy(zero_buf, cnt_bwd.at[pl.ds(voff, V_SLICE)])
    plsc.subcore_barrier()

    gbase = (s * NUM_CORES + c) * EDGES_PER_TILE

    def _hist(k, _):
        base = gbase + k * CHUNK
        ld_f = pltpu.async_copy(recv_hbm.at[pl.ds(base, CHUNK)], idx_f, sem)
        ld_b = pltpu.async_copy(send_hbm.at[pl.ds(base, CHUNK)], idx_b, sem)
        ld_f.wait()
        ld_b.wait()
        sc_f = pltpu.async_copy(ones_buf, cnt_fwd.at[idx_f], sem, add=True)
        sc_b = pltpu.async_copy(ones_buf, cnt_bwd.at[idx_b], sem, add=True)
        sc_f.wait()
        sc_b.wait()
        return 0
    lax.fori_loop(0, TILE_CHUNKS, _hist, 0)
    plsc.subcore_barrier()

    pltpu.sync_copy(cnt_fwd.at[pl.ds(voff, V_SLICE)], zero_buf)
    pltpu.sync_copy(zero_buf, part_hbm.at[pl.ds(c * 2 * V_PAD + voff, V_SLICE)])
    pltpu.sync_copy(cnt_bwd.at[pl.ds(voff, V_SLICE)], zero_buf)
    pltpu.sync_copy(zero_buf, part_hbm.at[pl.ds((c * 2 + 1) * V_PAD + voff, V_SLICE)])


@functools.partial(
    pl.kernel,
    out_type=(jax.ShapeDtypeStruct((EDGES,), jnp.float32),
              jax.ShapeDtypeStruct((EDGES,), jnp.float32)),
    mesh=_MESH,
    scratch_types=(
        pltpu.VMEM_SHARED((V_PAD,), jnp.float32),
        pltpu.VMEM_SHARED((V_PAD,), jnp.float32),
        pltpu.VMEM((CHUNK,), jnp.int32),
        pltpu.VMEM((CHUNK,), jnp.int32),
        pltpu.VMEM((CHUNK,), jnp.float32),
        pltpu.VMEM((CHUNK,), jnp.float32),
        pltpu.VMEM((V_SLICE,), jnp.float32),
        pltpu.VMEM((V_SLICE,), jnp.float32),
        pltpu.SemaphoreType.DMA,
    ),
)
def _gather_values(recv_hbm, send_hbm, part_hbm, fwd_hbm, bwd_hbm,
                   rec_fwd, rec_bwd, idx_f, idx_b, val_f, val_b,
                   pa_buf, pb_buf, sem):
    c = lax.axis_index("c")
    s = lax.axis_index("s")
    voff = s * V_SLICE

    def _recip_one(which, rec):
        pltpu.sync_copy(part_hbm.at[pl.ds(which * V_PAD + voff, V_SLICE)], pa_buf)
        pltpu.sync_copy(part_hbm.at[pl.ds((2 + which) * V_PAD + voff, V_SLICE)], pb_buf)

        def _r(i, _):
            tot = pa_buf[pl.ds(i * _LANES, _LANES)] + pb_buf[pl.ds(i * _LANES, _LANES)]
            pa_buf[pl.ds(i * _LANES, _LANES)] = 1.0 / tot
            return 0
        lax.fori_loop(0, V_SLICE // _LANES, _r, 0)
        pltpu.sync_copy(pa_buf, rec.at[pl.ds(voff, V_SLICE)])

    _recip_one(0, rec_fwd)
    _recip_one(1, rec_bwd)
    plsc.subcore_barrier()

    gbase = (s * NUM_CORES + c) * EDGES_PER_TILE

    def _gath(k, _):
        base = gbase + k * CHUNK
        ld_f = pltpu.async_copy(recv_hbm.at[pl.ds(base, CHUNK)], idx_f, sem)
        ld_b = pltpu.async_copy(send_hbm.at[pl.ds(base, CHUNK)], idx_b, sem)
        ld_f.wait()
        ld_b.wait()
        ga_f = pltpu.async_copy(rec_fwd.at[idx_f], val_f, sem)
        ga_b = pltpu.async_copy(rec_bwd.at[idx_b], val_b, sem)
        ga_f.wait()
        ga_b.wait()
        st_f = pltpu.async_copy(val_f, fwd_hbm.at[pl.ds(base, CHUNK)], sem)
        st_b = pltpu.async_copy(val_b, bwd_hbm.at[pl.ds(base, CHUNK)], sem)
        st_f.wait()
        st_b.wait()
        return 0
    lax.fori_loop(0, TILE_CHUNKS, _gath, 0)


def kernel(X):
    t = jnp.transpose(X)
    receivers = t[2]
    senders = t[0]
    partials = _count_partials(receivers, senders)
    fwd_values, bwd_values = _gather_values(receivers, senders, partials)
    message_indices = jnp.arange(EDGES, dtype=X.dtype)
    return (receivers, message_indices, fwd_values,
            senders, message_indices, bwd_values)


# speedup vs baseline: 1.1370x; 1.1370x over previous
import functools

import jax
import jax.numpy as jnp
from jax import lax
from jax.experimental import pallas as pl
from jax.experimental.pallas import tpu as pltpu
from jax.experimental.pallas import tpu_sc as plsc

VERTEXES = 100000
EDGES = 1600000

NUM_CORES = 2
NUM_SUBCORES = 16

V_SLICE = 6256
V_PAD = V_SLICE * NUM_SUBCORES

CHUNK = 10000
E_PER_TEC = EDGES // NUM_SUBCORES
N_CHUNKS = E_PER_TEC // CHUNK

_LANES = 16

_MESH = plsc.VectorSubcoreMesh(core_axis_name="c", subcore_axis_name="s",
                               num_cores=NUM_CORES, num_subcores=NUM_SUBCORES)


@functools.partial(
    pl.kernel,
    out_type=(jax.ShapeDtypeStruct((EDGES,), jnp.float32),
              jax.ShapeDtypeStruct((EDGES,), jnp.float32)),
    mesh=_MESH,
    scratch_types=(
        pltpu.VMEM_SHARED((V_PAD,), jnp.float32),
        pltpu.VMEM((E_PER_TEC,), jnp.int32),
        pltpu.VMEM((CHUNK,), jnp.float32),
        pltpu.VMEM((V_SLICE,), jnp.float32),
        pltpu.SemaphoreType.DMA,
    ),
)
def _incidence_values(recv_hbm, send_hbm, fwd_hbm, bwd_hbm,
                      hist, ids, val_buf, slice_buf, sem):
    c = lax.axis_index("c")
    s = lax.axis_index("s")

    def _role_body(col_hbm, out_hbm):
        def _fill(i, _):
            val_buf[pl.ds(i * _LANES, _LANES)] = jnp.full(
                (_LANES,), 1.0, jnp.float32)
            return 0
        lax.fori_loop(0, CHUNK // _LANES, _fill, 0)

        def _zero(i, _):
            slice_buf[pl.ds(i * _LANES, _LANES)] = jnp.zeros(
                (_LANES,), jnp.float32)
            return 0
        lax.fori_loop(0, V_SLICE // _LANES, _zero, 0)
        voff = s * V_SLICE
        pltpu.sync_copy(slice_buf, hist.at[pl.ds(voff, V_SLICE)])

        ebase = s * E_PER_TEC
        pltpu.sync_copy(col_hbm.at[pl.ds(ebase, E_PER_TEC)], ids)
        plsc.subcore_barrier()

        def _hist(k, _):
            pltpu.sync_copy(val_buf, hist.at[ids.at[pl.ds(k * CHUNK, CHUNK)]],
                            add=True)
            return 0
        lax.fori_loop(0, N_CHUNKS, _hist, 0)
        plsc.subcore_barrier()

        pltpu.sync_copy(hist.at[pl.ds(voff, V_SLICE)], slice_buf)

        def _recip(i, _):
            slice_buf[pl.ds(i * _LANES, _LANES)] = (
                1.0 / slice_buf[pl.ds(i * _LANES, _LANES)])
            return 0
        lax.fori_loop(0, V_SLICE // _LANES, _recip, 0)
        pltpu.sync_copy(slice_buf, hist.at[pl.ds(voff, V_SLICE)])
        plsc.subcore_barrier()

        def _gath(k, _):
            base = ebase + k * CHUNK
            pltpu.async_copy(hist.at[ids.at[pl.ds(k * CHUNK, CHUNK)]],
                             val_buf, sem).wait()
            pltpu.sync_copy(val_buf, out_hbm.at[pl.ds(base, CHUNK)])
            return 0
        lax.fori_loop(0, N_CHUNKS, _gath, 0)

    @pl.when(c == 0)
    def _():
        _role_body(recv_hbm, fwd_hbm)

    @pl.when(c == 1)
    def _():
        _role_body(send_hbm, bwd_hbm)


def kernel(X):
    t = jnp.transpose(X)
    receivers = t[2]
    senders = t[0]
    fwd_values, bwd_values = _incidence_values(receivers, senders)
    message_indices = jnp.arange(EDGES, dtype=X.dtype)
    return (receivers, message_indices, fwd_values,
            senders, message_indices, bwd_values)
